# Initial kernel scaffold; baseline (speedup 1.0000x reference)
import functools

import jax
import jax.numpy as jnp
from jax import lax
from jax.experimental import pallas as pl
from jax.experimental.pallas import tpu as pltpu
from jax.experimental.pallas import tpu_sc as plsc

B, V, N, M, IMG = 4, 8, 8192, 512, 224
PADW = 226 * 226 + 4
NC, NS, L = 2, 16, 16
NW = NC * NS
PTS_PER_TILE = (B * N) // NW
VECS = PTS_PER_TILE // L


def _tile_body(pxr, pyr, zf, pcx, pcy, pcz, maskp, bxn, byn, invc,
               ox, oy, oz,
               px_v, py_v, z_v, pcx_v, pcy_v, pcz_v,
               mask_v, bx_v, by_v, inv_v, ox_v, oy_v, oz_v):
    wid = lax.axis_index("c") * NS + lax.axis_index("s")
    batch = wid // (NW // B)
    base = wid * PTS_PER_TILE

    pltpu.sync_copy(pxr.at[pl.ds(base, PTS_PER_TILE)], px_v)
    pltpu.sync_copy(pyr.at[pl.ds(base, PTS_PER_TILE)], py_v)
    pltpu.sync_copy(zf.at[pl.ds(base, PTS_PER_TILE)], z_v)
    pltpu.sync_copy(pcx.at[pl.ds(base, PTS_PER_TILE)], pcx_v)
    pltpu.sync_copy(pcy.at[pl.ds(base, PTS_PER_TILE)], pcy_v)
    pltpu.sync_copy(pcz.at[pl.ds(base, PTS_PER_TILE)], pcz_v)
    pltpu.sync_copy(maskp.at[batch], mask_v)
    pltpu.sync_copy(bxn.at[batch], bx_v)
    pltpu.sync_copy(byn.at[batch], by_v)
    pltpu.sync_copy(invc.at[batch], inv_v)

    def point_vec(v, carry):
        s = v * L
        pxf = px_v[pl.ds(s, L)]
        pyf = py_v[pl.ds(s, L)]
        pxn = pxf / 224.0
        pyn = pyf / 224.0

        def cand(j, c):
            bd, bj = c
            jv = jnp.full((L,), j, dtype=jnp.int32)
            bx = plsc.load_gather(bx_v, [jv])
            by = plsc.load_gather(by_v, [jv])
            dx = pxn - bx
            dy = pyn - by
            d = dx * dx + dy * dy
            upd = d < bd
            return (jnp.where(upd, d, bd), jnp.where(upd, jv, bj))

        bd0 = jnp.full((L,), jnp.inf, dtype=jnp.float32)
        bj0 = jnp.zeros((L,), dtype=jnp.int32)
        _, bj = lax.fori_loop(0, M, cand, (bd0, bj0))

        nbx = plsc.load_gather(bx_v, [bj])
        nby = plsc.load_gather(by_v, [bj])

        pxi = pxf.astype(jnp.int32)
        pyi = pyf.astype(jnp.int32)
        xi = jnp.clip(pyi + 1, 0, 225)
        yi = jnp.clip(pxi + 1, 0, 225)
        mval = plsc.load_gather(mask_v, [xi * 226 + yi])
        flag = mval == 0.0

        zv = z_v[pl.ds(s, L)]
        b0 = (nbx * 224.0) * zv
        b1 = (nby * 224.0) * zv
        pc_vs = (pcx_v, pcy_v, pcz_v)
        o_vs = (ox_v, oy_v, oz_v)
        for cix in range(3):
            a0 = inv_v[pl.ds((0 * 3 + cix) * L, L)]
            a1 = inv_v[pl.ds((1 * 3 + cix) * L, L)]
            a2 = inv_v[pl.ds((2 * 3 + cix) * L, L)]
            a3 = inv_v[pl.ds((3 * 3 + cix) * L, L)]
            bc = b0 * a0 + b1 * a1 + zv * a2 + a3
            o_vs[cix][pl.ds(s, L)] = jnp.where(flag, bc, pc_vs[cix][pl.ds(s, L)])
        return carry

    lax.fori_loop(0, VECS, point_vec, 0)

    pltpu.sync_copy(ox_v, ox.at[pl.ds(base, PTS_PER_TILE)])
    pltpu.sync_copy(oy_v, oy.at[pl.ds(base, PTS_PER_TILE)])
    pltpu.sync_copy(oz_v, oz.at[pl.ds(base, PTS_PER_TILE)])


@functools.partial(
    pl.kernel,
    out_type=(
        jax.ShapeDtypeStruct((B * N,), jnp.float32),
        jax.ShapeDtypeStruct((B * N,), jnp.float32),
        jax.ShapeDtypeStruct((B * N,), jnp.float32),
    ),
    mesh=plsc.VectorSubcoreMesh(core_axis_name="c", subcore_axis_name="s"),
    scratch_types=[
        pltpu.VMEM((PTS_PER_TILE,), jnp.float32),
        pltpu.VMEM((PTS_PER_TILE,), jnp.float32),
        pltpu.VMEM((PTS_PER_TILE,), jnp.float32),
        pltpu.VMEM((PTS_PER_TILE,), jnp.float32),
        pltpu.VMEM((PTS_PER_TILE,), jnp.float32),
        pltpu.VMEM((PTS_PER_TILE,), jnp.float32),
        pltpu.VMEM((PADW,), jnp.float32),
        pltpu.VMEM((M,), jnp.float32),
        pltpu.VMEM((M,), jnp.float32),
        pltpu.VMEM((4 * 3 * L,), jnp.float32),
        pltpu.VMEM((PTS_PER_TILE,), jnp.float32),
        pltpu.VMEM((PTS_PER_TILE,), jnp.float32),
        pltpu.VMEM((PTS_PER_TILE,), jnp.float32),
    ],
)
def _sc_calibrate(*refs):
    _tile_body(*refs)


def kernel(pc, mask, bounds, view_id, inv_param, proj_fine, proj_finez):
    projv = lax.dynamic_index_in_dim(proj_fine, view_id, axis=1, keepdims=False)
    maskv = lax.dynamic_index_in_dim(mask, view_id, axis=1, keepdims=False)
    boundsv = lax.dynamic_index_in_dim(bounds, view_id, axis=1, keepdims=False)
    invv = lax.dynamic_index_in_dim(inv_param, view_id, axis=1, keepdims=False)
    zv = lax.dynamic_index_in_dim(proj_finez, view_id, axis=1, keepdims=False)

    pxr = jnp.round(projv[..., 0]).reshape(B * N)
    pyr = jnp.round(224.0 - projv[..., 1]).reshape(B * N)
    zf = zv.reshape(B * N)
    pcx = pc[..., 0].reshape(B * N)
    pcy = pc[..., 1].reshape(B * N)
    pcz = pc[..., 2].reshape(B * N)

    maskp = jnp.pad(maskv, ((0, 0), (1, 1), (1, 1))).reshape(B, 226 * 226)
    maskp = jnp.pad(maskp, ((0, 0), (0, PADW - 226 * 226)))

    bn = boundsv / 224.0
    bxn = bn[..., 0]
    byn = bn[..., 1]

    invc = jnp.broadcast_to(invv[:, :, :3, None], (B, 4, 3, L)).reshape(B, 4 * 3 * L)

    ox, oy, oz = _sc_calibrate(pxr, pyr, zf, pcx, pcy, pcz,
                               maskp, bxn, byn, invc)
    return jnp.stack([ox, oy, oz], axis=-1).reshape(B, N, 3)


# speedup vs baseline: 1.6240x; 1.6240x over previous
import functools

import jax
import jax.numpy as jnp
from jax import lax
from jax.experimental import pallas as pl
from jax.experimental.pallas import tpu as pltpu
from jax.experimental.pallas import tpu_sc as plsc

B, V, N, M, IMG = 4, 8, 8192, 512, 224
PADW = 226 * 226 + 4
NC, NS, L = 2, 16, 16
NW = NC * NS
PTS_PER_TILE = (B * N) // NW
VECS = PTS_PER_TILE // L


def _tile_body(pxr, pyr, zf, pcx, pcy, pcz, maskp, bxn, byn, invc,
               ox, oy, oz,
               px_v, py_v, z_v, pcx_v, pcy_v, pcz_v,
               mask_v, bx_v, by_v, inv_v, ox_v, oy_v, oz_v):
    wid = lax.axis_index("c") * NS + lax.axis_index("s")
    batch = wid // (NW // B)
    base = wid * PTS_PER_TILE

    pltpu.sync_copy(pxr.at[pl.ds(base, PTS_PER_TILE)], px_v)
    pltpu.sync_copy(pyr.at[pl.ds(base, PTS_PER_TILE)], py_v)
    pltpu.sync_copy(zf.at[pl.ds(base, PTS_PER_TILE)], z_v)
    pltpu.sync_copy(pcx.at[pl.ds(base, PTS_PER_TILE)], pcx_v)
    pltpu.sync_copy(pcy.at[pl.ds(base, PTS_PER_TILE)], pcy_v)
    pltpu.sync_copy(pcz.at[pl.ds(base, PTS_PER_TILE)], pcz_v)
    pltpu.sync_copy(maskp.at[batch], mask_v)
    pltpu.sync_copy(bxn.at[batch], bx_v)
    pltpu.sync_copy(byn.at[batch], by_v)
    pltpu.sync_copy(invc.at[batch], inv_v)

    def point_vec(v, carry):
        s = v * L
        pxf = px_v[pl.ds(s, L)]
        pyf = py_v[pl.ds(s, L)]
        pxn = pxf / 224.0
        pyn = pyf / 224.0

        def cand(j, c):
            bd, bj = c
            jv = jnp.full((L,), j, dtype=jnp.int32)
            bx = plsc.load_gather(bx_v, [jv])
            by = plsc.load_gather(by_v, [jv])
            dx = pxn - bx
            dy = pyn - by
            d = dx * dx + dy * dy
            upd = d < bd
            return (jnp.where(upd, d, bd), jnp.where(upd, jv, bj))

        bd0 = jnp.full((L,), jnp.inf, dtype=jnp.float32)
        bj0 = jnp.zeros((L,), dtype=jnp.int32)
        _, bj = lax.fori_loop(0, M, cand, (bd0, bj0))

        nbx = plsc.load_gather(bx_v, [bj])
        nby = plsc.load_gather(by_v, [bj])

        pxi = pxf.astype(jnp.int32)
        pyi = pyf.astype(jnp.int32)
        xi = jnp.clip(pyi + 1, 0, 225)
        yi = jnp.clip(pxi + 1, 0, 225)
        mval = plsc.load_gather(mask_v, [xi * 226 + yi])
        flag = mval == 0.0

        zv = z_v[pl.ds(s, L)]
        b0 = (nbx * 224.0) * zv
        b1 = (nby * 224.0) * zv
        pc_vs = (pcx_v, pcy_v, pcz_v)
        o_vs = (ox_v, oy_v, oz_v)
        for cix in range(3):
            a0 = inv_v[pl.ds((0 * 3 + cix) * L, L)]
            a1 = inv_v[pl.ds((1 * 3 + cix) * L, L)]
            a2 = inv_v[pl.ds((2 * 3 + cix) * L, L)]
            a3 = inv_v[pl.ds((3 * 3 + cix) * L, L)]
            bc = b0 * a0 + b1 * a1 + zv * a2 + a3
            o_vs[cix][pl.ds(s, L)] = jnp.where(flag, bc, pc_vs[cix][pl.ds(s, L)])
        return carry

    lax.fori_loop(0, VECS, point_vec, 0)

    pltpu.sync_copy(ox_v, ox.at[pl.ds(base, PTS_PER_TILE)])
    pltpu.sync_copy(oy_v, oy.at[pl.ds(base, PTS_PER_TILE)])
    pltpu.sync_copy(oz_v, oz.at[pl.ds(base, PTS_PER_TILE)])


@functools.partial(
    pl.kernel,
    out_type=(
        jax.ShapeDtypeStruct((B * N,), jnp.float32),
        jax.ShapeDtypeStruct((B * N,), jnp.float32),
        jax.ShapeDtypeStruct((B * N,), jnp.float32),
    ),
    mesh=plsc.VectorSubcoreMesh(core_axis_name="c", subcore_axis_name="s"),
    compiler_params=pltpu.CompilerParams(needs_layout_passes=False),
    scratch_types=[
        pltpu.VMEM((PTS_PER_TILE,), jnp.float32),
        pltpu.VMEM((PTS_PER_TILE,), jnp.float32),
        pltpu.VMEM((PTS_PER_TILE,), jnp.float32),
        pltpu.VMEM((PTS_PER_TILE,), jnp.float32),
        pltpu.VMEM((PTS_PER_TILE,), jnp.float32),
        pltpu.VMEM((PTS_PER_TILE,), jnp.float32),
        pltpu.VMEM((PADW,), jnp.float32),
        pltpu.VMEM((M,), jnp.float32),
        pltpu.VMEM((M,), jnp.float32),
        pltpu.VMEM((4 * 3 * L,), jnp.float32),
        pltpu.VMEM((PTS_PER_TILE,), jnp.float32),
        pltpu.VMEM((PTS_PER_TILE,), jnp.float32),
        pltpu.VMEM((PTS_PER_TILE,), jnp.float32),
    ],
)
def _sc_calibrate(*refs):
    _tile_body(*refs)


def kernel(pc, mask, bounds, view_id, inv_param, proj_fine, proj_finez):
    projv = lax.dynamic_index_in_dim(proj_fine, view_id, axis=1, keepdims=False)
    maskv = lax.dynamic_index_in_dim(mask, view_id, axis=1, keepdims=False)
    boundsv = lax.dynamic_index_in_dim(bounds, view_id, axis=1, keepdims=False)
    invv = lax.dynamic_index_in_dim(inv_param, view_id, axis=1, keepdims=False)
    zv = lax.dynamic_index_in_dim(proj_finez, view_id, axis=1, keepdims=False)

    pxr = jnp.round(projv[..., 0]).reshape(B * N)
    pyr = jnp.round(224.0 - projv[..., 1]).reshape(B * N)
    zf = zv.reshape(B * N)
    pcx = pc[..., 0].reshape(B * N)
    pcy = pc[..., 1].reshape(B * N)
    pcz = pc[..., 2].reshape(B * N)

    maskp = jnp.pad(maskv, ((0, 0), (1, 1), (1, 1))).reshape(B, 226 * 226)
    maskp = jnp.pad(maskp, ((0, 0), (0, PADW - 226 * 226)))

    bn = boundsv / 224.0
    bxn = bn[..., 0]
    byn = bn[..., 1]

    invc = jnp.broadcast_to(invv[:, :, :3, None], (B, 4, 3, L)).reshape(B, 4 * 3 * L)

    ox, oy, oz = _sc_calibrate(pxr, pyr, zf, pcx, pcy, pcz,
                               maskp, bxn, byn, invc)
    return jnp.stack([ox, oy, oz], axis=-1).reshape(B, N, 3)


# speedup vs baseline: 2.7052x; 1.6657x over previous
import functools

import jax
import jax.numpy as jnp
from jax import lax
from jax.experimental import pallas as pl
from jax.experimental.pallas import tpu as pltpu
from jax.experimental.pallas import tpu_sc as plsc

B, V, N, M, IMG = 4, 8, 8192, 512, 224
PADW = 226 * 226 + 4
NC, NS, L = 2, 16, 16
NW = NC * NS
PTS_PER_TILE = (B * N) // NW
VECS = PTS_PER_TILE // L


UNROLL = 4


def _tile_body(pxr, pyr, zf, pcx, pcy, pcz, maskp, bxn, byn, bxb, byb, invc,
               ox, oy, oz,
               px_v, py_v, z_v, pcx_v, pcy_v, pcz_v,
               mask_v, bx_v, by_v, bxb_v, byb_v, inv_v, ox_v, oy_v, oz_v):
    wid = lax.axis_index("c") * NS + lax.axis_index("s")
    batch = wid // (NW // B)
    base = wid * PTS_PER_TILE

    pltpu.sync_copy(pxr.at[pl.ds(base, PTS_PER_TILE)], px_v)
    pltpu.sync_copy(pyr.at[pl.ds(base, PTS_PER_TILE)], py_v)
    pltpu.sync_copy(zf.at[pl.ds(base, PTS_PER_TILE)], z_v)
    pltpu.sync_copy(pcx.at[pl.ds(base, PTS_PER_TILE)], pcx_v)
    pltpu.sync_copy(pcy.at[pl.ds(base, PTS_PER_TILE)], pcy_v)
    pltpu.sync_copy(pcz.at[pl.ds(base, PTS_PER_TILE)], pcz_v)
    pltpu.sync_copy(maskp.at[batch], mask_v)
    pltpu.sync_copy(bxn.at[batch], bx_v)
    pltpu.sync_copy(byn.at[batch], by_v)
    pltpu.sync_copy(bxb.at[batch], bxb_v)
    pltpu.sync_copy(byb.at[batch], byb_v)
    pltpu.sync_copy(invc.at[batch], inv_v)

    def point_vec(v, carry):
        s = v * L
        pxf = px_v[pl.ds(s, L)]
        pyf = py_v[pl.ds(s, L)]
        pxn = pxf / 224.0
        pyn = pyf / 224.0

        def cand_grp(g, c):
            bd, bj = c
            j0 = g * UNROLL
            pairs = []
            for u in range(UNROLL):
                j = j0 + u
                bx = bxb_v[pl.ds(j * L, L)]
                by = byb_v[pl.ds(j * L, L)]
                dx = pxn - bx
                dy = pyn - by
                d = dx * dx + dy * dy
                pairs.append((d, jnp.full((L,), j, dtype=jnp.int32)))
            while len(pairs) > 1:
                nxt = []
                for q in range(0, len(pairs), 2):
                    a, b = pairs[q], pairs[q + 1]
                    m = b[0] < a[0]
                    nxt.append((jnp.where(m, b[0], a[0]),
                                jnp.where(m, b[1], a[1])))
                pairs = nxt
            dg, jg = pairs[0]
            m = dg < bd
            return (jnp.where(m, dg, bd), jnp.where(m, jg, bj))

        bd0 = jnp.full((L,), jnp.inf, dtype=jnp.float32)
        bj0 = jnp.zeros((L,), dtype=jnp.int32)
        _, bj = lax.fori_loop(0, M // UNROLL, cand_grp, (bd0, bj0))

        nbx = plsc.load_gather(bx_v, [bj])
        nby = plsc.load_gather(by_v, [bj])

        pxi = pxf.astype(jnp.int32)
        pyi = pyf.astype(jnp.int32)
        xi = jnp.clip(pyi + 1, 0, 225)
        yi = jnp.clip(pxi + 1, 0, 225)
        mval = plsc.load_gather(mask_v, [xi * 226 + yi])
        flag = mval == 0.0

        zv = z_v[pl.ds(s, L)]
        b0 = (nbx * 224.0) * zv
        b1 = (nby * 224.0) * zv
        pc_vs = (pcx_v, pcy_v, pcz_v)
        o_vs = (ox_v, oy_v, oz_v)
        for cix in range(3):
            a0 = inv_v[pl.ds((0 * 3 + cix) * L, L)]
            a1 = inv_v[pl.ds((1 * 3 + cix) * L, L)]
            a2 = inv_v[pl.ds((2 * 3 + cix) * L, L)]
            a3 = inv_v[pl.ds((3 * 3 + cix) * L, L)]
            bc = b0 * a0 + b1 * a1 + zv * a2 + a3
            o_vs[cix][pl.ds(s, L)] = jnp.where(flag, bc, pc_vs[cix][pl.ds(s, L)])
        return carry

    lax.fori_loop(0, VECS, point_vec, 0)

    pltpu.sync_copy(ox_v, ox.at[pl.ds(base, PTS_PER_TILE)])
    pltpu.sync_copy(oy_v, oy.at[pl.ds(base, PTS_PER_TILE)])
    pltpu.sync_copy(oz_v, oz.at[pl.ds(base, PTS_PER_TILE)])


@functools.partial(
    pl.kernel,
    out_type=(
        jax.ShapeDtypeStruct((B * N,), jnp.float32),
        jax.ShapeDtypeStruct((B * N,), jnp.float32),
        jax.ShapeDtypeStruct((B * N,), jnp.float32),
    ),
    mesh=plsc.VectorSubcoreMesh(core_axis_name="c", subcore_axis_name="s"),
    compiler_params=pltpu.CompilerParams(needs_layout_passes=False),
    scratch_types=[
        pltpu.VMEM((PTS_PER_TILE,), jnp.float32),
        pltpu.VMEM((PTS_PER_TILE,), jnp.float32),
        pltpu.VMEM((PTS_PER_TILE,), jnp.float32),
        pltpu.VMEM((PTS_PER_TILE,), jnp.float32),
        pltpu.VMEM((PTS_PER_TILE,), jnp.float32),
        pltpu.VMEM((PTS_PER_TILE,), jnp.float32),
        pltpu.VMEM((PADW,), jnp.float32),
        pltpu.VMEM((M,), jnp.float32),
        pltpu.VMEM((M,), jnp.float32),
        pltpu.VMEM((M * L,), jnp.float32),
        pltpu.VMEM((M * L,), jnp.float32),
        pltpu.VMEM((4 * 3 * L,), jnp.float32),
        pltpu.VMEM((PTS_PER_TILE,), jnp.float32),
        pltpu.VMEM((PTS_PER_TILE,), jnp.float32),
        pltpu.VMEM((PTS_PER_TILE,), jnp.float32),
    ],
)
def _sc_calibrate(*refs):
    _tile_body(*refs)


def kernel(pc, mask, bounds, view_id, inv_param, proj_fine, proj_finez):
    projv = lax.dynamic_index_in_dim(proj_fine, view_id, axis=1, keepdims=False)
    maskv = lax.dynamic_index_in_dim(mask, view_id, axis=1, keepdims=False)
    boundsv = lax.dynamic_index_in_dim(bounds, view_id, axis=1, keepdims=False)
    invv = lax.dynamic_index_in_dim(inv_param, view_id, axis=1, keepdims=False)
    zv = lax.dynamic_index_in_dim(proj_finez, view_id, axis=1, keepdims=False)

    pxr = jnp.round(projv[..., 0]).reshape(B * N)
    pyr = jnp.round(224.0 - projv[..., 1]).reshape(B * N)
    zf = zv.reshape(B * N)
    pcx = pc[..., 0].reshape(B * N)
    pcy = pc[..., 1].reshape(B * N)
    pcz = pc[..., 2].reshape(B * N)

    maskp = jnp.pad(maskv, ((0, 0), (1, 1), (1, 1))).reshape(B, 226 * 226)
    maskp = jnp.pad(maskp, ((0, 0), (0, PADW - 226 * 226)))

    bn = boundsv / 224.0
    bxn = bn[..., 0]
    byn = bn[..., 1]
    bxb = jnp.broadcast_to(bxn[:, :, None], (B, M, L)).reshape(B, M * L)
    byb = jnp.broadcast_to(byn[:, :, None], (B, M, L)).reshape(B, M * L)

    invc = jnp.broadcast_to(invv[:, :, :3, None], (B, 4, 3, L)).reshape(B, 4 * 3 * L)

    ox, oy, oz = _sc_calibrate(pxr, pyr, zf, pcx, pcy, pcz,
                               maskp, bxn, byn, bxb, byb, invc)
    return jnp.stack([ox, oy, oz], axis=-1).reshape(B, N, 3)


# speedup vs baseline: 2.7665x; 1.0227x over previous
import functools

import jax
import jax.numpy as jnp
from jax import lax
from jax.experimental import pallas as pl
from jax.experimental.pallas import tpu as pltpu
from jax.experimental.pallas import tpu_sc as plsc

B, V, N, M, IMG = 4, 8, 8192, 512, 224
PADW = 226 * 226 + 4
NC, NS, L = 2, 16, 16
NW = NC * NS
PTS_PER_TILE = (B * N) // NW
VECS = PTS_PER_TILE // L


UNROLL = 8


def _tile_body(pxr, pyr, zf, pcx, pcy, pcz, maskp, bxn, byn, bxb, byb, invc,
               ox, oy, oz,
               px_v, py_v, z_v, pcx_v, pcy_v, pcz_v,
               mask_v, bx_v, by_v, bxb_v, byb_v, inv_v, ox_v, oy_v, oz_v):
    wid = lax.axis_index("c") * NS + lax.axis_index("s")
    batch = wid // (NW // B)
    base = wid * PTS_PER_TILE

    pltpu.sync_copy(pxr.at[pl.ds(base, PTS_PER_TILE)], px_v)
    pltpu.sync_copy(pyr.at[pl.ds(base, PTS_PER_TILE)], py_v)
    pltpu.sync_copy(zf.at[pl.ds(base, PTS_PER_TILE)], z_v)
    pltpu.sync_copy(pcx.at[pl.ds(base, PTS_PER_TILE)], pcx_v)
    pltpu.sync_copy(pcy.at[pl.ds(base, PTS_PER_TILE)], pcy_v)
    pltpu.sync_copy(pcz.at[pl.ds(base, PTS_PER_TILE)], pcz_v)
    pltpu.sync_copy(maskp.at[batch], mask_v)
    pltpu.sync_copy(bxn.at[batch], bx_v)
    pltpu.sync_copy(byn.at[batch], by_v)
    pltpu.sync_copy(bxb.at[batch], bxb_v)
    pltpu.sync_copy(byb.at[batch], byb_v)
    pltpu.sync_copy(invc.at[batch], inv_v)

    def point_vec(v, carry):
        s = v * L
        pxf = px_v[pl.ds(s, L)]
        pyf = py_v[pl.ds(s, L)]
        pxn = pxf / 224.0
        pyn = pyf / 224.0

        def cand_grp(g, c):
            bd, bj = c
            j0 = g * UNROLL
            pairs = []
            for u in range(UNROLL):
                j = j0 + u
                bx = bxb_v[pl.ds(j * L, L)]
                by = byb_v[pl.ds(j * L, L)]
                dx = pxn - bx
                dy = pyn - by
                d = dx * dx + dy * dy
                pairs.append((d, jnp.full((L,), j, dtype=jnp.int32)))
            while len(pairs) > 1:
                nxt = []
                for q in range(0, len(pairs), 2):
                    a, b = pairs[q], pairs[q + 1]
                    m = b[0] < a[0]
                    nxt.append((jnp.where(m, b[0], a[0]),
                                jnp.where(m, b[1], a[1])))
                pairs = nxt
            dg, jg = pairs[0]
            m = dg < bd
            return (jnp.where(m, dg, bd), jnp.where(m, jg, bj))

        bd0 = jnp.full((L,), jnp.inf, dtype=jnp.float32)
        bj0 = jnp.zeros((L,), dtype=jnp.int32)
        _, bj = lax.fori_loop(0, M // UNROLL, cand_grp, (bd0, bj0))

        nbx = plsc.load_gather(bx_v, [bj])
        nby = plsc.load_gather(by_v, [bj])

        pxi = pxf.astype(jnp.int32)
        pyi = pyf.astype(jnp.int32)
        xi = jnp.clip(pyi + 1, 0, 225)
        yi = jnp.clip(pxi + 1, 0, 225)
        mval = plsc.load_gather(mask_v, [xi * 226 + yi])
        flag = mval == 0.0

        zv = z_v[pl.ds(s, L)]
        b0 = (nbx * 224.0) * zv
        b1 = (nby * 224.0) * zv
        pc_vs = (pcx_v, pcy_v, pcz_v)
        o_vs = (ox_v, oy_v, oz_v)
        for cix in range(3):
            a0 = inv_v[pl.ds((0 * 3 + cix) * L, L)]
            a1 = inv_v[pl.ds((1 * 3 + cix) * L, L)]
            a2 = inv_v[pl.ds((2 * 3 + cix) * L, L)]
            a3 = inv_v[pl.ds((3 * 3 + cix) * L, L)]
            bc = b0 * a0 + b1 * a1 + zv * a2 + a3
            o_vs[cix][pl.ds(s, L)] = jnp.where(flag, bc, pc_vs[cix][pl.ds(s, L)])
        return carry

    lax.fori_loop(0, VECS, point_vec, 0)

    pltpu.sync_copy(ox_v, ox.at[pl.ds(base, PTS_PER_TILE)])
    pltpu.sync_copy(oy_v, oy.at[pl.ds(base, PTS_PER_TILE)])
    pltpu.sync_copy(oz_v, oz.at[pl.ds(base, PTS_PER_TILE)])


@functools.partial(
    pl.kernel,
    out_type=(
        jax.ShapeDtypeStruct((B * N,), jnp.float32),
        jax.ShapeDtypeStruct((B * N,), jnp.float32),
        jax.ShapeDtypeStruct((B * N,), jnp.float32),
    ),
    mesh=plsc.VectorSubcoreMesh(core_axis_name="c", subcore_axis_name="s"),
    compiler_params=pltpu.CompilerParams(needs_layout_passes=False),
    scratch_types=[
        pltpu.VMEM((PTS_PER_TILE,), jnp.float32),
        pltpu.VMEM((PTS_PER_TILE,), jnp.float32),
        pltpu.VMEM((PTS_PER_TILE,), jnp.float32),
        pltpu.VMEM((PTS_PER_TILE,), jnp.float32),
        pltpu.VMEM((PTS_PER_TILE,), jnp.float32),
        pltpu.VMEM((PTS_PER_TILE,), jnp.float32),
        pltpu.VMEM((PADW,), jnp.float32),
        pltpu.VMEM((M,), jnp.float32),
        pltpu.VMEM((M,), jnp.float32),
        pltpu.VMEM((M * L,), jnp.float32),
        pltpu.VMEM((M * L,), jnp.float32),
        pltpu.VMEM((4 * 3 * L,), jnp.float32),
        pltpu.VMEM((PTS_PER_TILE,), jnp.float32),
        pltpu.VMEM((PTS_PER_TILE,), jnp.float32),
        pltpu.VMEM((PTS_PER_TILE,), jnp.float32),
    ],
)
def _sc_calibrate(*refs):
    _tile_body(*refs)


def kernel(pc, mask, bounds, view_id, inv_param, proj_fine, proj_finez):
    projv = lax.dynamic_index_in_dim(proj_fine, view_id, axis=1, keepdims=False)
    maskv = lax.dynamic_index_in_dim(mask, view_id, axis=1, keepdims=False)
    boundsv = lax.dynamic_index_in_dim(bounds, view_id, axis=1, keepdims=False)
    invv = lax.dynamic_index_in_dim(inv_param, view_id, axis=1, keepdims=False)
    zv = lax.dynamic_index_in_dim(proj_finez, view_id, axis=1, keepdims=False)

    pxr = jnp.round(projv[..., 0]).reshape(B * N)
    pyr = jnp.round(224.0 - projv[..., 1]).reshape(B * N)
    zf = zv.reshape(B * N)
    pcx = pc[..., 0].reshape(B * N)
    pcy = pc[..., 1].reshape(B * N)
    pcz = pc[..., 2].reshape(B * N)

    maskp = jnp.pad(maskv, ((0, 0), (1, 1), (1, 1))).reshape(B, 226 * 226)
    maskp = jnp.pad(maskp, ((0, 0), (0, PADW - 226 * 226)))

    bn = boundsv / 224.0
    bxn = bn[..., 0]
    byn = bn[..., 1]
    bxb = jnp.broadcast_to(bxn[:, :, None], (B, M, L)).reshape(B, M * L)
    byb = jnp.broadcast_to(byn[:, :, None], (B, M, L)).reshape(B, M * L)

    invc = jnp.broadcast_to(invv[:, :, :3, None], (B, 4, 3, L)).reshape(B, 4 * 3 * L)

    ox, oy, oz = _sc_calibrate(pxr, pyr, zf, pcx, pcy, pcz,
                               maskp, bxn, byn, bxb, byb, invc)
    return jnp.stack([ox, oy, oz], axis=-1).reshape(B, N, 3)
